# baseline (speedup 1.0000x reference)
import numpy as np
import jax
import jax.numpy as jnp
from jax import lax
from jax.experimental import pallas as pl
from jax.experimental.pallas import tpu as pltpu
from jax.experimental.pallas import tpu_sc as plsc

_D = 32
_ND = 34
_B = 16
_NROW = _B * _D * _D
_NOUT = _B * _ND * _ND
_NC = 2
_NS = 16

_S = np.arange(_D) + (np.arange(_D) >= 5) + (np.arange(_D) >= 15)
_CIDX = (_S[:, None] * _ND + _S[None, :]).reshape(-1).astype(np.int32)

_LROW = tuple(tuple(int(_S[8 * case + r]) for r in range(8)) for case in range(4))


def _sc_body(rho_hbm, cidx_hbm, zeros_hbm, out_hbm,
             in0, in1, band0, band1, cidx_buf,
             isem0, isem1, osem0, osem1):
    wid = lax.axis_index("s") * _NC + lax.axis_index("c")
    b = wid // 2
    ih = wid % 2
    srcb = b * 1024 + ih * 512

    pltpu.sync_copy(cidx_hbm, cidx_buf)
    pltpu.sync_copy(zeros_hbm, band0)
    pltpu.sync_copy(zeros_hbm, band1)

    @pl.when(ih == 0)
    def _():
        pltpu.sync_copy(band0, out_hbm.at[b, pl.ds(5 * _ND, _ND), :])
        pltpu.sync_copy(band0, out_hbm.at[b, pl.ds(16 * _ND, _ND), :])

    def in_start(t, half, buf, sem):
        pltpu.make_async_copy(
            rho_hbm.at[pl.ds(srcb + t * 32 + half * 16, 16), :], buf,
            sem).start()

    def in_wait(buf, sem):
        pltpu.make_async_copy(rho_hbm.at[pl.ds(0, 16), :], buf, sem).wait()

    def out_start(t, buf, sem):
        i = ih * 16 + t
        si = i + (i >= 5).astype(jnp.int32) + (i >= 15).astype(jnp.int32)
        pltpu.make_async_copy(
            buf, out_hbm.at[b, pl.ds(si * _ND, _ND), :], sem).start()

    def out_wait(buf, sem):
        pltpu.make_async_copy(
            buf, out_hbm.at[0, pl.ds(0, _ND), :], sem).wait()

    def fill(case, ibuf, bbuf):
        lrow = _LROW[case]
        ris = [jnp.full((16,), lr, jnp.int32) for lr in lrow]

        @plsc.parallel_loop(0, _D * _D // 16, step=1, unroll=4)
        def _(c):
            ci = cidx_buf[pl.ds(c * 16, 16)]
            for r in range(8):
                v = ibuf[(case % 2) * 8 + r, pl.ds(c * 16, 16)]
                plsc.store_scatter(bbuf, [ris[r], ci], v)

    in_start(0, 0, in0, isem0)

    def hband(hb, carry):
        for x, bbuf, osem in ((0, band0, osem0), (1, band1, osem1)):
            t = 2 * hb + x
            in_wait(in0, isem0)
            in_start(t, 1, in1, isem1)

            @pl.when(hb >= 1)
            def _():
                out_wait(bbuf, osem)

            fill(0, in0, bbuf)
            fill(1, in0, bbuf)

            in_wait(in1, isem1)
            if x == 0:
                in_start(t + 1, 0, in0, isem0)
            else:
                @pl.when(hb < 7)
                def _():
                    in_start(t + 1, 0, in0, isem0)

            fill(2, in1, bbuf)
            fill(3, in1, bbuf)
            out_start(t, bbuf, osem)
        return carry

    lax.fori_loop(0, 8, hband, 0)
    out_wait(band0, osem0)
    out_wait(band1, osem1)


def kernel(rho):
    rho2 = rho.reshape(_NROW, _D * _D)
    zeros = jnp.zeros((_ND, _ND * _ND), jnp.float32)
    mesh = plsc.VectorSubcoreMesh(core_axis_name="c", subcore_axis_name="s",
                                  num_cores=_NC)
    run = pl.kernel(
        _sc_body,
        mesh=mesh,
        out_type=jax.ShapeDtypeStruct((_B, _ND * _ND, _ND * _ND), jnp.float32),
        scratch_types=[
            pltpu.VMEM((16, _D * _D), jnp.float32),
            pltpu.VMEM((16, _D * _D), jnp.float32),
            pltpu.VMEM((_ND, _ND * _ND), jnp.float32),
            pltpu.VMEM((_ND, _ND * _ND), jnp.float32),
            pltpu.VMEM((_D * _D,), jnp.int32),
            pltpu.SemaphoreType.DMA,
            pltpu.SemaphoreType.DMA,
            pltpu.SemaphoreType.DMA,
            pltpu.SemaphoreType.DMA,
        ],
        compiler_params=pltpu.CompilerParams(needs_layout_passes=False,
                                             use_tc_tiling_on_sc=False),
    )
    return run(rho2, jnp.asarray(_CIDX), zeros)


# speedup vs baseline: 1.7467x; 1.7467x over previous
import numpy as np
import jax
import jax.numpy as jnp
from jax import lax
from jax.experimental import pallas as pl
from jax.experimental.pallas import tpu as pltpu
from jax.experimental.pallas import tpu_sc as plsc

_D = 32
_ND = 34
_B = 16
_NROW = _B * _D * _D
_NOUT = _B * _ND * _ND
_NC = 2
_NS = 16

_S = np.arange(_D) + (np.arange(_D) >= 5) + (np.arange(_D) >= 15)
_CIDX = (_S[:, None] * _ND + _S[None, :]).reshape(-1).astype(np.int32)

_LROW = tuple(tuple(int(_S[8 * case + r]) for r in range(8)) for case in range(4))


def _sc_body(rho_hbm, cidx_hbm, zeros_hbm, out_hbm,
             in0, in1, band0, band1, cidx_buf,
             isem0, isem1, osem0, osem1):
    wid = lax.axis_index("s") * _NC + lax.axis_index("c")
    b = wid // 2
    ih = wid % 2
    srcb = b * 1024 + ih * 512

    pltpu.sync_copy(cidx_hbm, cidx_buf)
    pltpu.sync_copy(zeros_hbm, band0)
    pltpu.sync_copy(zeros_hbm, band1)

    @pl.when(ih == 0)
    def _():
        pltpu.sync_copy(band0, out_hbm.at[pl.ds(b * _ND * _ND + 5 * _ND, _ND), :])
        pltpu.sync_copy(band0, out_hbm.at[pl.ds(b * _ND * _ND + 16 * _ND, _ND), :])

    def in_start(t, half, buf, sem):
        pltpu.make_async_copy(
            rho_hbm.at[pl.ds(srcb + t * 32 + half * 16, 16), :], buf,
            sem).start()

    def in_wait(buf, sem):
        pltpu.make_async_copy(rho_hbm.at[pl.ds(0, 16), :], buf, sem).wait()

    def out_start(t, buf, sem):
        i = ih * 16 + t
        si = i + (i >= 5).astype(jnp.int32) + (i >= 15).astype(jnp.int32)
        pltpu.make_async_copy(
            buf, out_hbm.at[pl.ds(b * _ND * _ND + si * _ND, _ND), :], sem).start()

    def out_wait(buf, sem):
        pltpu.make_async_copy(
            buf, out_hbm.at[pl.ds(0, _ND), :], sem).wait()

    def fill(case, ibuf, bbuf):
        lrow = _LROW[case]
        ris = [jnp.full((16,), lr, jnp.int32) for lr in lrow]

        @plsc.parallel_loop(0, _D * _D // 16, step=1, unroll=8)
        def _(c):
            ci = cidx_buf[pl.ds(c * 16, 16)]
            for r in range(8):
                v = ibuf[(case % 2) * 8 + r, pl.ds(c * 16, 16)]
                plsc.store_scatter(bbuf, [ris[r], ci], v)

    in_start(0, 0, in0, isem0)

    def hband(hb, carry):
        for x, bbuf, osem in ((0, band0, osem0), (1, band1, osem1)):
            t = 2 * hb + x
            in_wait(in0, isem0)
            in_start(t, 1, in1, isem1)

            @pl.when(hb >= 1)
            def _():
                out_wait(bbuf, osem)

            fill(0, in0, bbuf)
            fill(1, in0, bbuf)

            in_wait(in1, isem1)
            if x == 0:
                in_start(t + 1, 0, in0, isem0)
            else:
                @pl.when(hb < 7)
                def _():
                    in_start(t + 1, 0, in0, isem0)

            fill(2, in1, bbuf)
            fill(3, in1, bbuf)
            out_start(t, bbuf, osem)
        return carry

    lax.fori_loop(0, 8, hband, 0)
    out_wait(band0, osem0)
    out_wait(band1, osem1)


def kernel(rho):
    rho2 = rho.reshape(_NROW, _D * _D)
    zeros = jnp.zeros((_ND, _ND * _ND), jnp.float32)
    mesh = plsc.VectorSubcoreMesh(core_axis_name="c", subcore_axis_name="s",
                                  num_cores=_NC)
    run = pl.kernel(
        _sc_body,
        mesh=mesh,
        out_type=jax.ShapeDtypeStruct((_NOUT, _ND * _ND), jnp.float32),
        scratch_types=[
            pltpu.VMEM((16, _D * _D), jnp.float32),
            pltpu.VMEM((16, _D * _D), jnp.float32),
            pltpu.VMEM((_ND, _ND * _ND), jnp.float32),
            pltpu.VMEM((_ND, _ND * _ND), jnp.float32),
            pltpu.VMEM((_D * _D,), jnp.int32),
            pltpu.SemaphoreType.DMA,
            pltpu.SemaphoreType.DMA,
            pltpu.SemaphoreType.DMA,
            pltpu.SemaphoreType.DMA,
        ],
        compiler_params=pltpu.CompilerParams(needs_layout_passes=False,
                                             use_tc_tiling_on_sc=False),
    )
    out = run(rho2, jnp.asarray(_CIDX), zeros)
    return out.reshape(_B, _ND * _ND, _ND * _ND)


# speedup vs baseline: 2.1527x; 1.2324x over previous
import numpy as np
import jax
import jax.numpy as jnp
from jax import lax
from jax.experimental import pallas as pl
from jax.experimental.pallas import tpu as pltpu
from jax.experimental.pallas import tpu_sc as plsc

_D = 32
_ND = 34
_B = 16
_NROW = _B * _D * _D
_NDP = 1160
_NOUT = _B * _NDP
_NC = 2
_NS = 16

_S = np.arange(_D) + (np.arange(_D) >= 5) + (np.arange(_D) >= 15)
_CIDX = (_S[:, None] * _ND + _S[None, :]).reshape(-1).astype(np.int32)

_LROW = tuple(tuple(int(_S[8 * case + r]) for r in range(8)) for case in range(4))


def _sc_body(rho_hbm, cidx_hbm, zeros_hbm, out_hbm,
             in0, in1, band0, band1, cidx_buf,
             isem0, isem1, osem0, osem1):
    wid = lax.axis_index("s") * _NC + lax.axis_index("c")
    b = wid // 2
    ih = wid % 2
    srcb = b * 1024 + ih * 512

    pltpu.sync_copy(cidx_hbm, cidx_buf)
    pltpu.sync_copy(zeros_hbm, band0)
    pltpu.sync_copy(zeros_hbm, band1)

    @pl.when(ih == 0)
    def _():
        pltpu.sync_copy(band0, out_hbm.at[pl.ds(b * _NDP + 5 * _ND, _ND), :])
        pltpu.sync_copy(band0, out_hbm.at[pl.ds(b * _NDP + 16 * _ND, _ND), :])

    def in_start(t, half, buf, sem):
        pltpu.make_async_copy(
            rho_hbm.at[pl.ds(srcb + t * 32 + half * 16, 16), :], buf,
            sem).start()

    def in_wait(buf, sem):
        pltpu.make_async_copy(rho_hbm.at[pl.ds(0, 16), :], buf, sem).wait()

    def out_start(t, buf, sem):
        i = ih * 16 + t
        si = i + (i >= 5).astype(jnp.int32) + (i >= 15).astype(jnp.int32)
        pltpu.make_async_copy(
            buf, out_hbm.at[pl.ds(b * _NDP + si * _ND, _ND), :], sem).start()

    def out_wait(buf, sem):
        pltpu.make_async_copy(
            buf, out_hbm.at[pl.ds(0, _ND), :], sem).wait()

    def fill(case, ibuf, bbuf):
        lrow = _LROW[case]
        ris = [jnp.full((16,), lr, jnp.int32) for lr in lrow]

        @plsc.parallel_loop(0, _D * _D // 16, step=1, unroll=4)
        def _(c):
            ci = cidx_buf[pl.ds(c * 16, 16)]
            for r in range(8):
                v = ibuf[(case % 2) * 8 + r, pl.ds(c * 16, 16)]
                plsc.store_scatter(bbuf, [ris[r], ci], v)

    in_start(0, 0, in0, isem0)

    def hband(hb, carry):
        for x, bbuf, osem in ((0, band0, osem0), (1, band1, osem1)):
            t = 2 * hb + x
            in_wait(in0, isem0)
            in_start(t, 1, in1, isem1)

            @pl.when(hb >= 1)
            def _():
                out_wait(bbuf, osem)

            fill(0, in0, bbuf)
            fill(1, in0, bbuf)

            in_wait(in1, isem1)
            if x == 0:
                in_start(t + 1, 0, in0, isem0)
            else:
                @pl.when(hb < 7)
                def _():
                    in_start(t + 1, 0, in0, isem0)

            fill(2, in1, bbuf)
            fill(3, in1, bbuf)
            out_start(t, bbuf, osem)
        return carry

    lax.fori_loop(0, 8, hband, 0)
    out_wait(band0, osem0)
    out_wait(band1, osem1)


def kernel(rho):
    rho2 = rho.reshape(_NROW, _D * _D)
    zeros = jnp.zeros((_ND, _ND * _ND), jnp.float32)
    mesh = plsc.VectorSubcoreMesh(core_axis_name="c", subcore_axis_name="s",
                                  num_cores=_NC)
    run = pl.kernel(
        _sc_body,
        mesh=mesh,
        out_type=jax.ShapeDtypeStruct((_NOUT, _ND * _ND), jnp.float32),
        scratch_types=[
            pltpu.VMEM((16, _D * _D), jnp.float32),
            pltpu.VMEM((16, _D * _D), jnp.float32),
            pltpu.VMEM((_ND, _ND * _ND), jnp.float32),
            pltpu.VMEM((_ND, _ND * _ND), jnp.float32),
            pltpu.VMEM((_D * _D,), jnp.int32),
            pltpu.SemaphoreType.DMA,
            pltpu.SemaphoreType.DMA,
            pltpu.SemaphoreType.DMA,
            pltpu.SemaphoreType.DMA,
        ],
        compiler_params=pltpu.CompilerParams(needs_layout_passes=False,
                                             use_tc_tiling_on_sc=False),
    )
    out = run(rho2, jnp.asarray(_CIDX), zeros)
    return out.reshape(_B, _NDP, _ND * _ND)[:, :_ND * _ND, :]
